# Initial kernel scaffold; baseline (speedup 1.0000x reference)
import jax
import jax.numpy as jnp
from jax.experimental import pallas as pl


def kernel(log_classifications, unique_ids):
    raise NotImplementedError("write your pallas kernel here")


# speedup vs baseline: 3.1880x; 3.1880x over previous
import jax
import jax.numpy as jnp
from jax.experimental import pallas as pl
from jax.experimental.pallas import tpu as pltpu

_POS_WEIGHT = 30.0 / 10.0
_ROWS, _COLS = 128, 8192
_BLK_ROWS = 16
_GRID = _ROWS // _BLK_ROWS


def _loss_body(x_ref, lo_ref, hi_ref, out_ref, acc_ref):
    step = pl.program_id(0)

    @pl.when(step == 0)
    def _init():
        acc_ref[0] = 0.0
        acc_ref[1] = 0.0

    x = x_ref[...]
    lo = lo_ref[...]
    hi = hi_ref[...]
    keep = jnp.logical_not((lo == -2) & (hi == -1))
    tgt = (lo == -1) & (hi == -1)
    sp = jnp.maximum(x, 0.0) + jnp.log1p(jnp.exp(-jnp.abs(x)))
    per = jnp.where(tgt, _POS_WEIGHT * (sp - x), sp)
    per = jnp.where(keep, per, 0.0)
    acc_ref[0] += jnp.sum(per)
    acc_ref[1] += jnp.sum(keep.astype(jnp.float32))

    @pl.when(step == _GRID - 1)
    def _fin():
        c = acc_ref[1]
        out_ref[0, 0] = acc_ref[0] / (c * c)


def kernel(log_classifications, unique_ids):
    id_lo = unique_ids.astype(jnp.int32)
    id_hi = (unique_ids >> 32).astype(jnp.int32)
    out = pl.pallas_call(
        _loss_body,
        grid=(_GRID,),
        in_specs=[
            pl.BlockSpec((_BLK_ROWS, _COLS), lambda i: (i, jnp.int32(0))),
            pl.BlockSpec((_BLK_ROWS, _COLS), lambda i: (i, jnp.int32(0))),
            pl.BlockSpec((_BLK_ROWS, _COLS), lambda i: (i, jnp.int32(0))),
        ],
        out_specs=pl.BlockSpec(
            (1, 1), lambda i: (jnp.int32(0), jnp.int32(0)), memory_space=pltpu.SMEM
        ),
        out_shape=jax.ShapeDtypeStruct((1, 1), jnp.float32),
        scratch_shapes=[pltpu.SMEM((2,), jnp.float32)],
    )(log_classifications, id_lo, id_hi)
    return out[0, 0]
